# Initial kernel scaffold; baseline (speedup 1.0000x reference)
import jax
import jax.numpy as jnp
from jax.experimental import pallas as pl


def kernel(emb_k, emb_v, length_q):
    raise NotImplementedError("write your pallas kernel here")


# speedup vs baseline: 2.4934x; 2.4934x over previous
import functools

import jax
import jax.numpy as jnp
from jax import lax
from jax.experimental import pallas as pl
from jax.experimental.pallas import tpu as pltpu
from jax.experimental.pallas import tpu_sc as plsc

D = 256
KMAX = 64
L = 512
BT = 2 * L

_mesh = plsc.VectorSubcoreMesh(core_axis_name="c", subcore_axis_name="s")


@functools.partial(
    pl.kernel,
    mesh=_mesh,
    out_type=(
        jax.ShapeDtypeStruct((L * L, D), jnp.float32),
        jax.ShapeDtypeStruct((L * L, D), jnp.float32),
    ),
    scratch_types=[
        pltpu.VMEM((64,), jnp.int32),
        pltpu.VMEM((64, D), jnp.float32),
        pltpu.VMEM_SHARED((BT, D), jnp.float32),
        pltpu.VMEM_SHARED((BT, D), jnp.float32),
        pltpu.SemaphoreType.DMA,
    ],
    compiler_params=pltpu.CompilerParams(use_tc_tiling_on_sc=False),
)
def _rel_pos_sc(embk, embv, outk, outv, idx_v, stage_v, bk, bv, sem):
    s = lax.axis_index("s")
    c = lax.axis_index("c")

    base_t = s * 64
    for ch in range(4):
        tvec = lax.iota(jnp.int32, 16) + (base_t + ch * 16)
        idx_v[pl.ds(ch * 16, 16)] = jnp.clip(tvec - (L - 1 - KMAX), 0, 2 * KMAX)
    pltpu.async_copy(embk.at[idx_v], stage_v, sem).wait()
    pltpu.sync_copy(stage_v, bk.at[pl.ds(base_t, 64)])
    pltpu.async_copy(embv.at[idx_v], stage_v, sem).wait()
    pltpu.sync_copy(stage_v, bv.at[pl.ds(base_t, 64)])
    plsc.subcore_barrier()

    wid = s * 2 + c
    for r in range(16):
        i = wid * 16 + r
        start = (L - 1) - i
        pltpu.sync_copy(bk.at[pl.ds(start, L)], outk.at[pl.ds(i * L, L)])
        pltpu.sync_copy(bv.at[pl.ds(start, L)], outv.at[pl.ds(i * L, L)])


def kernel(emb_k, emb_v, length_q):
    del length_q
    ok, ov = _rel_pos_sc(emb_k, emb_v)
    return ok.reshape(L, L, D), ov.reshape(L, L, D)
